# baseline (speedup 1.0000x reference)
import jax
import jax.numpy as jnp
from jax.experimental import pallas as pl
from jax.experimental.pallas import tpu as pltpu

_BATCH = 16384
_DIM = 2048
_NCHUNK = 8
_CHUNK = _BATCH // _NCHUNK


def _copy_body(h_ref, o_ref, sems):
    for k in range(_NCHUNK):
        pltpu.make_async_copy(
            h_ref.at[pl.ds(k * _CHUNK, _CHUNK), :],
            o_ref.at[pl.ds(k * _CHUNK, _CHUNK), :],
            sems.at[k],
        ).start()
    for k in range(_NCHUNK):
        pltpu.make_async_copy(
            h_ref.at[pl.ds(k * _CHUNK, _CHUNK), :],
            o_ref.at[pl.ds(k * _CHUNK, _CHUNK), :],
            sems.at[k],
        ).wait()


def kernel(h):
    return pl.pallas_call(
        _copy_body,
        in_specs=[pl.BlockSpec(memory_space=pl.ANY)],
        out_specs=pl.BlockSpec(memory_space=pl.ANY),
        out_shape=jax.ShapeDtypeStruct((_BATCH, _DIM), jnp.float32),
        scratch_shapes=[pltpu.SemaphoreType.DMA((_NCHUNK,))],
    )(h)


# speedup vs baseline: 48.1241x; 48.1241x over previous
import jax
import jax.numpy as jnp
from jax.experimental import pallas as pl

_BATCH = 16384
_DIM = 2048
_BLOCK_ROWS = 512


def _copy_body(h_ref, o_ref):
    o_ref[...] = h_ref[...]


def kernel(h):
    grid = (_BATCH // _BLOCK_ROWS,)
    return pl.pallas_call(
        _copy_body,
        grid=grid,
        in_specs=[pl.BlockSpec((_BLOCK_ROWS, _DIM), lambda i: (i, 0))],
        out_specs=pl.BlockSpec((_BLOCK_ROWS, _DIM), lambda i: (i, 0)),
        out_shape=jax.ShapeDtypeStruct((_BATCH, _DIM), jnp.float32),
    )(h)
